# Initial kernel scaffold; baseline (speedup 1.0000x reference)
import jax
import jax.numpy as jnp
from jax.experimental import pallas as pl


def kernel(candidate_pts, src_keypts, tgt_pts_xyz, tgt_deep_feat_pts):
    raise NotImplementedError("write your pallas kernel here")


# speedup vs baseline: 39.8514x; 39.8514x over previous
import jax
import jax.numpy as jnp
from jax import lax
from jax.experimental import pallas as pl
from jax.experimental.pallas import tpu as pltpu
from jax.experimental.pallas import tpu_sc as plsc

RADIUS = 2.0
R2 = RADIUS * RADIUS
KNN = 32
DFEAT = 32
DOUT = DFEAT + 3
NLANE = 16
NCORES = 2
NSUB = 16
NWORKERS = NCORES * NSUB
GCH = 128


DPAD = 48


def _table_body(xyz_ref, feat_ref, out_ref):
    blk = xyz_ref.shape[0]
    out_ref[:] = jnp.concatenate(
        [xyz_ref[:] * (1.0 / RADIUS), feat_ref[:],
         jnp.zeros((blk, DPAD - DOUT), jnp.float32)], axis=1)


def _build_table(xyz2, feat2):
    m = xyz2.shape[0]
    g = 8
    return pl.pallas_call(
        _table_body,
        out_shape=jax.ShapeDtypeStruct((m, DPAD), jnp.float32),
        grid=(g,),
        in_specs=[
            pl.BlockSpec((m // g, 3), lambda i: (i, 0)),
            pl.BlockSpec((m // g, DFEAT), lambda i: (i, 0)),
        ],
        out_specs=pl.BlockSpec((m // g, DPAD), lambda i: (i, 0)),
    )(xyz2, feat2)


def _make_sc_kernel(n, nq, qpt, nbatch):
    nstep = n // NLANE
    nchunk = qpt * KNN // GCH
    qrow = GCH // KNN

    def body(qt, xyzt, table, out, xs, ys, zs, qxs, qys, qzs, idxq, idx2d,
             rows, sem):
        wid = lax.axis_index("s") * NCORES + lax.axis_index("c")
        b = wid // (NWORKERS // nbatch)
        qbase = wid * qpt
        pltpu.sync_copy(qt.at[0, pl.ds(qbase, qpt)], qxs)
        pltpu.sync_copy(qt.at[1, pl.ds(qbase, qpt)], qys)
        pltpu.sync_copy(qt.at[2, pl.ds(qbase, qpt)], qzs)
        pltpu.sync_copy(xyzt.at[3 * b + 0], xs)
        pltpu.sync_copy(xyzt.at[3 * b + 1], ys)
        pltpu.sync_copy(xyzt.at[3 * b + 2], zs)
        rowbase = b * n
        lanes = lax.iota(jnp.int32, NLANE)
        dnums = lax.GatherDimensionNumbers(
            offset_dims=(), collapsed_slice_dims=(0,), start_index_map=(0,))

        def _splat(vec, j):
            sel = jnp.full((NLANE, 1), j, jnp.int32)
            return lax.gather(vec, sel, dnums, (1,),
                              mode=lax.GatherScatterMode.PROMISE_IN_BOUNDS)

        def per_group(g, carry):
            qxv = qxs[pl.ds(g * NLANE, NLANE)]
            qyv = qys[pl.ds(g * NLANE, NLANE)]
            qzv = qzs[pl.ds(g * NLANE, NLANE)]

            def per_query(j, carry2):
                qx = _splat(qxv, j)
                qy = _splat(qyv, j)
                qz = _splat(qzv, j)

                def cond(c):
                    i, cnt = c
                    return jnp.logical_and(cnt < KNN, i < nstep)

                def step(c):
                    i, cnt = c
                    sl = pl.ds(i * NLANE, NLANE)
                    dx = xs[sl] - qx
                    dy = ys[sl] - qy
                    dz = zs[sl] - qz
                    d = dx * dx + dy * dy + dz * dz
                    m = d <= R2
                    plsc.store_compressed(
                        idxq.at[pl.ds(cnt, NLANE)],
                        lanes + (i * NLANE + rowbase), mask=m)
                    return i + jnp.int32(1), cnt + jnp.sum(m.astype(jnp.int32))

                _, cnt = lax.while_loop(
                    cond, step, (jnp.int32(0), jnp.int32(0)))
                v0 = idxq[pl.ds(0, NLANE)]
                v1 = idxq[pl.ds(NLANE, NLANE)]
                first = jnp.where(cnt > 0, _splat(v0, 0),
                                  jnp.full((NLANE,), rowbase + (n - 1),
                                           jnp.int32))
                o0 = jnp.where(lanes < cnt, v0, first)
                o1 = jnp.where(lanes + NLANE < cnt, v1, first)
                qi = g * NLANE + j
                r = qi // qrow
                c0 = (qi % qrow) * KNN
                idx2d[r, pl.ds(c0, NLANE)] = o0
                idx2d[r, pl.ds(c0 + NLANE, NLANE)] = o1
                return carry2

            return lax.fori_loop(0, NLANE, per_query, carry)

        lax.fori_loop(0, qpt // NLANE, per_group, 0)
        copies = [
            pltpu.async_copy(table.at[idx2d.at[j]],
                             rows.at[pl.ds(j * GCH, GCH)], sem)
            for j in range(nchunk)
        ]
        for cp in copies:
            cp.wait()
        pltpu.sync_copy(rows, out.at[pl.ds(wid * (qpt * KNN), qpt * KNN)])

    mesh = plsc.VectorSubcoreMesh(
        core_axis_name="c", subcore_axis_name="s",
        num_cores=NCORES, num_subcores=NSUB)
    return pl.kernel(
        body,
        out_type=jax.ShapeDtypeStruct((nq * KNN, DPAD), jnp.float32),
        mesh=mesh,
        compiler_params=pltpu.CompilerParams(
            needs_layout_passes=False, use_tc_tiling_on_sc=False),
        scratch_types=[
            pltpu.VMEM((n,), jnp.float32),
            pltpu.VMEM((n,), jnp.float32),
            pltpu.VMEM((n,), jnp.float32),
            pltpu.VMEM((qpt,), jnp.float32),
            pltpu.VMEM((qpt,), jnp.float32),
            pltpu.VMEM((qpt,), jnp.float32),
            pltpu.VMEM((KNN + NLANE,), jnp.int32),
            pltpu.VMEM((nchunk, GCH), jnp.int32),
            pltpu.VMEM((qpt * KNN, DPAD), jnp.float32),
            pltpu.SemaphoreType.DMA,
        ],
    )


def kernel(candidate_pts, src_keypts, tgt_pts_xyz, tgt_deep_feat_pts):
    del src_keypts
    b, ktop, c, _ = candidate_pts.shape
    s = ktop * c
    n = tgt_pts_xyz.shape[1]
    nq = b * s
    qpt = nq // NWORKERS

    qt = candidate_pts.reshape(nq, 3).T
    xyzt = tgt_pts_xyz.transpose(0, 2, 1).reshape(b * 3, n)
    table = _build_table(
        tgt_pts_xyz.reshape(b * n, 3),
        tgt_deep_feat_pts.reshape(b * n, DFEAT))
    out = _make_sc_kernel(n, nq, qpt, b)(qt, xyzt, table)
    return out[:, :DOUT].reshape(b, ktop, c, KNN, DOUT)
